# baseline (speedup 1.0000x reference)
import jax
import jax.numpy as jnp
from jax import lax
from jax.experimental import pallas as pl
from jax.experimental.pallas import tpu as pltpu
from jax.experimental.pallas import tpu_sc as plsc

_B, _S, _H = 1, 2048, 768
_F = 3072
_E = 8
_EPS = 1e-12
_T = 256
_G = _S * 2 // _T + _E
_P = _G * _T
_NC, _NS = 2, 16
_NW = _NC * _NS
_TPW = _S // _NW
_SQRT1_2 = 0.7071067811865476


def _router_body(x_ref, rw_ref, rb_ref,
                 pos0_ref, pos1_ref, w0_ref, w1_ref, te_ref):
    x = x_ref[...]
    logits = lax.dot_general(
        x.astype(jnp.bfloat16), rw_ref[...].astype(jnp.bfloat16),
        (((1,), (1,)), ((), ())),
        preferred_element_type=jnp.float32) + rb_ref[...]
    iota_e = lax.broadcasted_iota(jnp.int32, logits.shape, 1)
    m0 = jnp.max(logits, axis=-1, keepdims=True)
    e0 = jnp.min(jnp.where(logits >= m0, iota_e, _E), axis=-1, keepdims=True)
    masked = jnp.where(iota_e == e0, -jnp.inf, logits)
    m1 = jnp.max(masked, axis=-1, keepdims=True)
    e1 = jnp.min(jnp.where(masked >= m1, iota_e, _E), axis=-1, keepdims=True)
    w0 = 1.0 / (1.0 + jnp.exp(m1 - m0))
    w0_ref[...] = w0
    w1_ref[...] = 1.0 - w0
    sel = ((iota_e == e0) | (iota_e == e1)).astype(jnp.bfloat16)
    tri = (lax.broadcasted_iota(jnp.int32, (_S, _S), 1)
           < lax.broadcasted_iota(jnp.int32, (_S, _S), 0)).astype(jnp.bfloat16)
    rank = lax.dot_general(tri, sel, (((1,), (0,)), ((), ())),
                           preferred_element_type=jnp.float32)
    count = jnp.sum(sel.astype(jnp.float32), axis=0, keepdims=True)
    pc = ((count.astype(jnp.int32) + _T - 1) // _T) * _T
    trie = (lax.broadcasted_iota(jnp.int32, (_E, _E), 0)
            < lax.broadcasted_iota(jnp.int32, (_E, _E), 1)).astype(jnp.float32)
    pstart = lax.dot_general(pc.astype(jnp.float32), trie,
                             (((1,), (0,)), ((), ())),
                             preferred_element_type=jnp.float32,
                             precision=lax.Precision.HIGHEST)
    rank0 = jnp.sum(jnp.where(iota_e == e0, rank, 0.0), axis=1, keepdims=True)
    rank1 = jnp.sum(jnp.where(iota_e == e1, rank, 0.0), axis=1, keepdims=True)
    ps0 = jnp.sum(jnp.where(iota_e == e0, pstart, 0.0), axis=1, keepdims=True)
    ps1 = jnp.sum(jnp.where(iota_e == e1, pstart, 0.0), axis=1, keepdims=True)
    pos0_ref[...] = (ps0 + rank0).astype(jnp.int32)
    pos1_ref[...] = (ps1 + rank1).astype(jnp.int32)
    psi = pstart.astype(jnp.int32)
    n_used = jnp.sum(pc, axis=1, keepdims=True) // _T
    iota_g = lax.broadcasted_iota(jnp.int32, (_G, _E), 0)
    gT = jnp.minimum(iota_g, n_used - 1) * _T
    te = jnp.sum((jnp.broadcast_to(psi, (_G, _E)) <= gT).astype(jnp.int32),
                 axis=1, keepdims=True) - 1
    te = jnp.clip(te, 0, _E - 1)
    iota_e_g = lax.broadcasted_iota(jnp.int32, (_G, _E), 1)
    used_b = jnp.broadcast_to((count > 0).astype(jnp.int32), (_G, _E))
    ordv = jnp.sum(jnp.where(iota_e_g < te, used_b, 0),
                   axis=1, keepdims=True)
    parity = ordv % 2
    te_prev = jnp.concatenate(
        [jnp.full((1, 1), -1, jnp.int32), te[:-1]], axis=0)
    is_start = (te != te_prev).astype(jnp.int32)
    nxt = jnp.min(jnp.where((used_b == 1) & (iota_e_g > te), iota_e_g, _E),
                  axis=1, keepdims=True)
    te_ref[...] = jnp.concatenate(
        [te, parity, is_start, nxt,
         jnp.broadcast_to(n_used, (8, 1))], axis=0)


def _router(flat, router_w, router_b):
    return pl.pallas_call(
        _router_body,
        grid=(1,),
        in_specs=[
            pl.BlockSpec((_S, _H), lambda i: (0, 0)),
            pl.BlockSpec((_E, _H), lambda i: (0, 0)),
            pl.BlockSpec((1, _E), lambda i: (0, 0)),
        ],
        out_specs=[
            pl.BlockSpec((_S, 1), lambda i: (0, 0)),
            pl.BlockSpec((_S, 1), lambda i: (0, 0)),
            pl.BlockSpec((_S, 1), lambda i: (0, 0)),
            pl.BlockSpec((_S, 1), lambda i: (0, 0)),
            pl.BlockSpec((4 * _G + 8, 1), lambda i: (0, 0)),
        ],
        out_shape=[
            jax.ShapeDtypeStruct((_S, 1), jnp.int32),
            jax.ShapeDtypeStruct((_S, 1), jnp.int32),
            jax.ShapeDtypeStruct((_S, 1), jnp.float32),
            jax.ShapeDtypeStruct((_S, 1), jnp.float32),
            jax.ShapeDtypeStruct((4 * _G + 8, 1), jnp.int32),
        ],
    )(flat, router_w, router_b.reshape(1, _E))


def _sc_scatter_body(flat_hbm, pos0_hbm, pos1_hbm, xs_hbm,
                     rows_v, i0_v, i1_v, sem):
    wid = lax.axis_index("s") * _NC + lax.axis_index("c")
    base = wid * _TPW
    pltpu.sync_copy(flat_hbm.at[pl.ds(base, _TPW)], rows_v)
    pltpu.sync_copy(pos0_hbm.at[pl.ds(base, _TPW)], i0_v)
    pltpu.sync_copy(pos1_hbm.at[pl.ds(base, _TPW)], i1_v)
    pltpu.async_copy(rows_v, xs_hbm.at[i0_v], sem).wait()
    pltpu.async_copy(rows_v, xs_hbm.at[i1_v], sem).wait()


def _sc_scatter(flat, pos0, pos1):
    mesh = plsc.VectorSubcoreMesh(core_axis_name="c", subcore_axis_name="s")
    return pl.kernel(
        _sc_scatter_body,
        out_type=jax.ShapeDtypeStruct((_P, _H), jnp.float32),
        mesh=mesh,
        scratch_types=[
            pltpu.VMEM((_TPW, _H), jnp.float32),
            pltpu.VMEM((_TPW,), jnp.int32),
            pltpu.VMEM((_TPW,), jnp.int32),
            pltpu.SemaphoreType.DMA,
        ],
    )(flat, pos0, pos1)


def _ffn_body(m_ref, xs_ref, b1_ref, b2_ref, w1_hbm, w2_hbm, y_ref,
              w1buf, w2buf, w1bb, w2bb, semw):
    g = pl.program_id(0)
    e = m_ref[g]
    par = m_ref[_G + g]
    st = m_ref[2 * _G + g]
    nxe = m_ref[3 * _G + g]
    nu = m_ref[4 * _G]

    @pl.when(g == 0)
    def _():
        pltpu.make_async_copy(w1_hbm.at[e], w1buf.at[par], semw.at[par]).start()
        pltpu.make_async_copy(w2_hbm.at[e], w2buf.at[par], semw.at[par]).start()

    @pl.when(st == 1)
    def _():
        pltpu.make_async_copy(w1_hbm.at[e], w1buf.at[par], semw.at[par]).wait()
        pltpu.make_async_copy(w2_hbm.at[e], w2buf.at[par], semw.at[par]).wait()
        w1bb[...] = w1buf[par].astype(jnp.bfloat16)
        w2bb[...] = w2buf[par].astype(jnp.bfloat16)

    @pl.when((st == 1) & (nxe < _E))
    def _():
        pltpu.make_async_copy(
            w1_hbm.at[nxe], w1buf.at[1 - par], semw.at[1 - par]).start()
        pltpu.make_async_copy(
            w2_hbm.at[nxe], w2buf.at[1 - par], semw.at[1 - par]).start()

    @pl.when(g < nu)
    def _():
        xb = xs_ref[...].astype(jnp.bfloat16)
        h1 = lax.dot_general(xb, w1bb[...], (((1,), (1,)), ((), ())),
                             preferred_element_type=jnp.float32) + b1_ref[0]
        h1 = 0.5 * h1 * (1.0 + lax.erf(h1 * _SQRT1_2))
        y_ref[...] = lax.dot_general(
            h1.astype(jnp.bfloat16), w2bb[...], (((1,), (1,)), ((), ())),
            preferred_element_type=jnp.float32) + b2_ref[0]


def _ffn(te_flat, xs, b1r, b2r, W1, W2):
    grid_spec = pltpu.PrefetchScalarGridSpec(
        num_scalar_prefetch=1,
        grid=(_G,),
        in_specs=[
            pl.BlockSpec((_T, _H), lambda g, m: (g, 0)),
            pl.BlockSpec((1, 1, _F), lambda g, m: (m[g], 0, 0)),
            pl.BlockSpec((1, 1, _H), lambda g, m: (m[g], 0, 0)),
            pl.BlockSpec(memory_space=pltpu.MemorySpace.HBM),
            pl.BlockSpec(memory_space=pltpu.MemorySpace.HBM),
        ],
        out_specs=pl.BlockSpec((_T, _H), lambda g, m: (g, 0)),
        scratch_shapes=[
            pltpu.VMEM((2, _F, _H), jnp.float32),
            pltpu.VMEM((2, _H, _F), jnp.float32),
            pltpu.VMEM((_F, _H), jnp.bfloat16),
            pltpu.VMEM((_H, _F), jnp.bfloat16),
            pltpu.SemaphoreType.DMA((2,)),
        ],
    )
    return pl.pallas_call(
        _ffn_body,
        grid_spec=grid_spec,
        out_shape=jax.ShapeDtypeStruct((_P, _H), jnp.float32),
        compiler_params=pltpu.CompilerParams(
            vmem_limit_bytes=120 * 1024 * 1024),
    )(te_flat, xs, b1r, b2r, W1, W2)


def _sc_gather_body(y_hbm, pos0_hbm, pos1_hbm, y0_hbm, y1_hbm,
                    rows_v, i_v, sem):
    wid = lax.axis_index("s") * _NC + lax.axis_index("c")
    base = wid * _TPW
    pltpu.sync_copy(pos0_hbm.at[pl.ds(base, _TPW)], i_v)
    pltpu.async_copy(y_hbm.at[i_v], rows_v, sem).wait()
    pltpu.sync_copy(rows_v, y0_hbm.at[pl.ds(base, _TPW)])
    pltpu.sync_copy(pos1_hbm.at[pl.ds(base, _TPW)], i_v)
    pltpu.async_copy(y_hbm.at[i_v], rows_v, sem).wait()
    pltpu.sync_copy(rows_v, y1_hbm.at[pl.ds(base, _TPW)])


def _sc_gather(y, pos0, pos1):
    mesh = plsc.VectorSubcoreMesh(core_axis_name="c", subcore_axis_name="s")
    return pl.kernel(
        _sc_gather_body,
        out_type=[
            jax.ShapeDtypeStruct((_S, _H), jnp.float32),
            jax.ShapeDtypeStruct((_S, _H), jnp.float32),
        ],
        mesh=mesh,
        scratch_types=[
            pltpu.VMEM((_TPW, _H), jnp.float32),
            pltpu.VMEM((_TPW,), jnp.int32),
            pltpu.SemaphoreType.DMA,
        ],
    )(y, pos0, pos1)


def _combine_body(x_ref, y0_ref, y1_ref, w0_ref, w1_ref, g_ref, b_ref,
                  out_ref):
    u = (x_ref[...] + w0_ref[...] * y0_ref[...] + w1_ref[...] * y1_ref[...])
    mu = jnp.mean(u, axis=-1, keepdims=True)
    var = jnp.mean((u - mu) ** 2, axis=-1, keepdims=True)
    out_ref[...] = (u - mu) * lax.rsqrt(var + _EPS) * g_ref[...] + b_ref[...]


def _combine(flat, y0, y1, w0, w1, ln_g, ln_b):
    tn = 256
    return pl.pallas_call(
        _combine_body,
        grid=(_S // tn,),
        in_specs=[
            pl.BlockSpec((tn, _H), lambda t: (t, 0)),
            pl.BlockSpec((tn, _H), lambda t: (t, 0)),
            pl.BlockSpec((tn, _H), lambda t: (t, 0)),
            pl.BlockSpec((tn, 1), lambda t: (t, 0)),
            pl.BlockSpec((tn, 1), lambda t: (t, 0)),
            pl.BlockSpec((1, _H), lambda t: (0, 0)),
            pl.BlockSpec((1, _H), lambda t: (0, 0)),
        ],
        out_specs=pl.BlockSpec((tn, _H), lambda t: (t, 0)),
        out_shape=jax.ShapeDtypeStruct((_S, _H), jnp.float32),
    )(flat, y0, y1, w0, w1, ln_g.reshape(1, _H), ln_b.reshape(1, _H))


def kernel(hidden_states, router_w, router_b, W1, b1, W2, b2, ln_g, ln_b):
    flat = hidden_states.reshape(_S, _H)
    pos0_2d, pos1_2d, w0, w1, te = _router(flat, router_w, router_b)
    pos0 = pos0_2d.reshape(_S)
    pos1 = pos1_2d.reshape(_S)
    xs = _sc_scatter(flat, pos0, pos1)
    y = _ffn(te.reshape(4 * _G + 8), xs, b1.reshape(_E, 1, _F),
             b2.reshape(_E, 1, _H), W1, W2)
    y0, y1 = _sc_gather(y, pos0, pos1)
    out = _combine(flat, y0, y1, w0, w1, ln_g, ln_b)
    return out.reshape(_B, _S, _H)


# speedup vs baseline: 1.0567x; 1.0567x over previous
import jax
import jax.numpy as jnp
from jax import lax
from jax.experimental import pallas as pl
from jax.experimental.pallas import tpu as pltpu
from jax.experimental.pallas import tpu_sc as plsc

_B, _S, _H = 1, 2048, 768
_F = 3072
_E = 8
_EPS = 1e-12
_T = 256
_G = _S * 2 // _T + _E
_P = _G * _T
_NC, _NS = 2, 16
_NW = _NC * _NS
_TPW = _S // _NW
_SQRT1_2 = 0.7071067811865476


def _router_body(x_ref, rw_ref, rb_ref,
                 pos0_ref, pos1_ref, w0_ref, w1_ref, te_ref):
    x = x_ref[...]
    logits = lax.dot_general(
        x.astype(jnp.bfloat16), rw_ref[...].astype(jnp.bfloat16),
        (((1,), (1,)), ((), ())),
        preferred_element_type=jnp.float32) + rb_ref[...]
    iota_e = lax.broadcasted_iota(jnp.int32, logits.shape, 1)
    m0 = jnp.max(logits, axis=-1, keepdims=True)
    e0 = jnp.min(jnp.where(logits >= m0, iota_e, _E), axis=-1, keepdims=True)
    masked = jnp.where(iota_e == e0, -jnp.inf, logits)
    m1 = jnp.max(masked, axis=-1, keepdims=True)
    e1 = jnp.min(jnp.where(masked >= m1, iota_e, _E), axis=-1, keepdims=True)
    w0 = 1.0 / (1.0 + jnp.exp(m1 - m0))
    w0_ref[...] = w0
    w1_ref[...] = 1.0 - w0
    sel = ((iota_e == e0) | (iota_e == e1)).astype(jnp.bfloat16)
    tri = (lax.broadcasted_iota(jnp.int32, (_S, _S), 1)
           < lax.broadcasted_iota(jnp.int32, (_S, _S), 0)).astype(jnp.bfloat16)
    rank = lax.dot_general(tri, sel, (((1,), (0,)), ((), ())),
                           preferred_element_type=jnp.float32)
    count = jnp.sum(sel.astype(jnp.float32), axis=0, keepdims=True)
    pc = ((count.astype(jnp.int32) + _T - 1) // _T) * _T
    trie = (lax.broadcasted_iota(jnp.int32, (_E, _E), 0)
            < lax.broadcasted_iota(jnp.int32, (_E, _E), 1)).astype(jnp.float32)
    pstart = lax.dot_general(pc.astype(jnp.float32), trie,
                             (((1,), (0,)), ((), ())),
                             preferred_element_type=jnp.float32,
                             precision=lax.Precision.HIGHEST)
    rank0 = jnp.sum(jnp.where(iota_e == e0, rank, 0.0), axis=1, keepdims=True)
    rank1 = jnp.sum(jnp.where(iota_e == e1, rank, 0.0), axis=1, keepdims=True)
    ps0 = jnp.sum(jnp.where(iota_e == e0, pstart, 0.0), axis=1, keepdims=True)
    ps1 = jnp.sum(jnp.where(iota_e == e1, pstart, 0.0), axis=1, keepdims=True)
    pos0_ref[...] = (ps0 + rank0).astype(jnp.int32)
    pos1_ref[...] = (ps1 + rank1).astype(jnp.int32)
    psi = pstart.astype(jnp.int32)
    n_used = jnp.sum(pc, axis=1, keepdims=True) // _T
    iota_g = lax.broadcasted_iota(jnp.int32, (_G, _E), 0)
    gT = jnp.minimum(iota_g, n_used - 1) * _T
    te = jnp.sum((jnp.broadcast_to(psi, (_G, _E)) <= gT).astype(jnp.int32),
                 axis=1, keepdims=True) - 1
    te = jnp.clip(te, 0, _E - 1)
    iota_e_g = lax.broadcasted_iota(jnp.int32, (_G, _E), 1)
    used_b = jnp.broadcast_to((count > 0).astype(jnp.int32), (_G, _E))
    ordv = jnp.sum(jnp.where(iota_e_g < te, used_b, 0),
                   axis=1, keepdims=True)
    parity = ordv % 2
    te_prev = jnp.concatenate(
        [jnp.full((1, 1), -1, jnp.int32), te[:-1]], axis=0)
    is_start = (te != te_prev).astype(jnp.int32)
    nxt = jnp.min(jnp.where((used_b == 1) & (iota_e_g > te), iota_e_g, _E),
                  axis=1, keepdims=True)
    te_ref[...] = jnp.concatenate(
        [te, parity, is_start, nxt,
         jnp.broadcast_to(n_used, (8, 1))], axis=0)


def _router(flat, router_w, router_b):
    return pl.pallas_call(
        _router_body,
        grid=(1,),
        in_specs=[
            pl.BlockSpec((_S, _H), lambda i: (0, 0)),
            pl.BlockSpec((_E, _H), lambda i: (0, 0)),
            pl.BlockSpec((1, _E), lambda i: (0, 0)),
        ],
        out_specs=[
            pl.BlockSpec((_S, 1), lambda i: (0, 0)),
            pl.BlockSpec((_S, 1), lambda i: (0, 0)),
            pl.BlockSpec((_S, 1), lambda i: (0, 0)),
            pl.BlockSpec((_S, 1), lambda i: (0, 0)),
            pl.BlockSpec((4 * _G + 8, 1), lambda i: (0, 0)),
        ],
        out_shape=[
            jax.ShapeDtypeStruct((_S, 1), jnp.int32),
            jax.ShapeDtypeStruct((_S, 1), jnp.int32),
            jax.ShapeDtypeStruct((_S, 1), jnp.float32),
            jax.ShapeDtypeStruct((_S, 1), jnp.float32),
            jax.ShapeDtypeStruct((4 * _G + 8, 1), jnp.int32),
        ],
    )(flat, router_w, router_b.reshape(1, _E))


def _sc_scatter_body(flat_hbm, pos0_hbm, pos1_hbm, xs_hbm,
                     rows_v, i0_v, i1_v, sem, sem2):
    wid = lax.axis_index("s") * _NC + lax.axis_index("c")
    base = wid * _TPW
    cr = pltpu.async_copy(flat_hbm.at[pl.ds(base, _TPW)], rows_v, sem)
    c0 = pltpu.async_copy(pos0_hbm.at[pl.ds(base, _TPW)], i0_v, sem2)
    c1 = pltpu.async_copy(pos1_hbm.at[pl.ds(base, _TPW)], i1_v, sem2)
    cr.wait()
    c0.wait()
    c1.wait()
    s0 = pltpu.async_copy(rows_v, xs_hbm.at[i0_v], sem)
    s1 = pltpu.async_copy(rows_v, xs_hbm.at[i1_v], sem2)
    s0.wait()
    s1.wait()


def _sc_scatter(flat, pos0, pos1):
    mesh = plsc.VectorSubcoreMesh(core_axis_name="c", subcore_axis_name="s")
    return pl.kernel(
        _sc_scatter_body,
        out_type=jax.ShapeDtypeStruct((_P, _H), jnp.float32),
        mesh=mesh,
        scratch_types=[
            pltpu.VMEM((_TPW, _H), jnp.float32),
            pltpu.VMEM((_TPW,), jnp.int32),
            pltpu.VMEM((_TPW,), jnp.int32),
            pltpu.SemaphoreType.DMA,
            pltpu.SemaphoreType.DMA,
        ],
    )(flat, pos0, pos1)


def _ffn_body(m_ref, xs_ref, b1_ref, b2_ref, w1_hbm, w2_hbm, y_ref,
              w1buf, w2buf, semw):
    g = pl.program_id(0)
    e = m_ref[g]
    par = m_ref[_G + g]
    st = m_ref[2 * _G + g]
    nxe = m_ref[3 * _G + g]
    nu = m_ref[4 * _G]

    @pl.when(g == 0)
    def _():
        pltpu.make_async_copy(w1_hbm.at[e], w1buf.at[par], semw.at[par]).start()
        pltpu.make_async_copy(w2_hbm.at[e], w2buf.at[par], semw.at[par]).start()

    @pl.when(st == 1)
    def _():
        pltpu.make_async_copy(w1_hbm.at[e], w1buf.at[par], semw.at[par]).wait()
        pltpu.make_async_copy(w2_hbm.at[e], w2buf.at[par], semw.at[par]).wait()

    @pl.when((st == 1) & (nxe < _E))
    def _():
        pltpu.make_async_copy(
            w1_hbm.at[nxe], w1buf.at[1 - par], semw.at[1 - par]).start()
        pltpu.make_async_copy(
            w2_hbm.at[nxe], w2buf.at[1 - par], semw.at[1 - par]).start()

    @pl.when(g < nu)
    def _():
        xb = xs_ref[...].astype(jnp.bfloat16)
        h1 = lax.dot_general(xb, w1buf[par].astype(jnp.bfloat16),
                             (((1,), (1,)), ((), ())),
                             preferred_element_type=jnp.float32) + b1_ref[0]
        h1 = 0.5 * h1 * (1.0 + lax.erf(h1 * _SQRT1_2))
        y_ref[...] = lax.dot_general(
            h1.astype(jnp.bfloat16), w2buf[par].astype(jnp.bfloat16),
            (((1,), (1,)), ((), ())),
            preferred_element_type=jnp.float32) + b2_ref[0]


def _ffn(te_flat, xs, b1r, b2r, W1, W2):
    grid_spec = pltpu.PrefetchScalarGridSpec(
        num_scalar_prefetch=1,
        grid=(_G,),
        in_specs=[
            pl.BlockSpec((_T, _H), lambda g, m: (g, 0)),
            pl.BlockSpec((1, 1, _F), lambda g, m: (m[g], 0, 0)),
            pl.BlockSpec((1, 1, _H), lambda g, m: (m[g], 0, 0)),
            pl.BlockSpec(memory_space=pltpu.MemorySpace.HBM),
            pl.BlockSpec(memory_space=pltpu.MemorySpace.HBM),
        ],
        out_specs=pl.BlockSpec((_T, _H), lambda g, m: (g, 0)),
        scratch_shapes=[
            pltpu.VMEM((2, _F, _H), jnp.float32),
            pltpu.VMEM((2, _H, _F), jnp.float32),
            pltpu.SemaphoreType.DMA((2,)),
        ],
    )
    return pl.pallas_call(
        _ffn_body,
        grid_spec=grid_spec,
        out_shape=jax.ShapeDtypeStruct((_P, _H), jnp.float32),
        compiler_params=pltpu.CompilerParams(
            vmem_limit_bytes=120 * 1024 * 1024),
    )(te_flat, xs, b1r, b2r, W1, W2)


def _sc_gather_body(y_hbm, pos0_hbm, pos1_hbm, y0_hbm, y1_hbm,
                    rows0_v, rows1_v, i0_v, i1_v, sem, sem2):
    wid = lax.axis_index("s") * _NC + lax.axis_index("c")
    base = wid * _TPW
    c0 = pltpu.async_copy(pos0_hbm.at[pl.ds(base, _TPW)], i0_v, sem)
    c1 = pltpu.async_copy(pos1_hbm.at[pl.ds(base, _TPW)], i1_v, sem2)
    c0.wait()
    c1.wait()
    g0 = pltpu.async_copy(y_hbm.at[i0_v], rows0_v, sem)
    g1 = pltpu.async_copy(y_hbm.at[i1_v], rows1_v, sem2)
    g0.wait()
    g1.wait()
    s0 = pltpu.async_copy(rows0_v, y0_hbm.at[pl.ds(base, _TPW)], sem)
    s1 = pltpu.async_copy(rows1_v, y1_hbm.at[pl.ds(base, _TPW)], sem2)
    s0.wait()
    s1.wait()


def _sc_gather(y, pos0, pos1):
    mesh = plsc.VectorSubcoreMesh(core_axis_name="c", subcore_axis_name="s")
    return pl.kernel(
        _sc_gather_body,
        out_type=[
            jax.ShapeDtypeStruct((_S, _H), jnp.float32),
            jax.ShapeDtypeStruct((_S, _H), jnp.float32),
        ],
        mesh=mesh,
        scratch_types=[
            pltpu.VMEM((_TPW, _H), jnp.float32),
            pltpu.VMEM((_TPW, _H), jnp.float32),
            pltpu.VMEM((_TPW,), jnp.int32),
            pltpu.VMEM((_TPW,), jnp.int32),
            pltpu.SemaphoreType.DMA,
            pltpu.SemaphoreType.DMA,
        ],
    )(y, pos0, pos1)


def _combine_body(x_ref, y0_ref, y1_ref, w0_ref, w1_ref, g_ref, b_ref,
                  out_ref):
    u = (x_ref[...] + w0_ref[...] * y0_ref[...] + w1_ref[...] * y1_ref[...])
    mu = jnp.mean(u, axis=-1, keepdims=True)
    var = jnp.mean((u - mu) ** 2, axis=-1, keepdims=True)
    out_ref[...] = (u - mu) * lax.rsqrt(var + _EPS) * g_ref[...] + b_ref[...]


def _combine(flat, y0, y1, w0, w1, ln_g, ln_b):
    tn = 256
    return pl.pallas_call(
        _combine_body,
        grid=(_S // tn,),
        in_specs=[
            pl.BlockSpec((tn, _H), lambda t: (t, 0)),
            pl.BlockSpec((tn, _H), lambda t: (t, 0)),
            pl.BlockSpec((tn, _H), lambda t: (t, 0)),
            pl.BlockSpec((tn, 1), lambda t: (t, 0)),
            pl.BlockSpec((tn, 1), lambda t: (t, 0)),
            pl.BlockSpec((1, _H), lambda t: (0, 0)),
            pl.BlockSpec((1, _H), lambda t: (0, 0)),
        ],
        out_specs=pl.BlockSpec((tn, _H), lambda t: (t, 0)),
        out_shape=jax.ShapeDtypeStruct((_S, _H), jnp.float32),
    )(flat, y0, y1, w0, w1, ln_g.reshape(1, _H), ln_b.reshape(1, _H))


def kernel(hidden_states, router_w, router_b, W1, b1, W2, b2, ln_g, ln_b):
    flat = hidden_states.reshape(_S, _H)
    pos0_2d, pos1_2d, w0, w1, te = _router(flat, router_w, router_b)
    pos0 = pos0_2d.reshape(_S)
    pos1 = pos1_2d.reshape(_S)
    xs = _sc_scatter(flat, pos0, pos1)
    y = _ffn(te.reshape(4 * _G + 8), xs, b1.reshape(_E, 1, _F),
             b2.reshape(_E, 1, _H), W1, W2)
    y0, y1 = _sc_gather(y, pos0, pos1)
    out = _combine(flat, y0, y1, w0, w1, ln_g, ln_b)
    return out.reshape(_B, _S, _H)


# speedup vs baseline: 1.1514x; 1.0896x over previous
import jax
import jax.numpy as jnp
from jax import lax
from jax.experimental import pallas as pl
from jax.experimental.pallas import tpu as pltpu
from jax.experimental.pallas import tpu_sc as plsc

_B, _S, _H = 1, 2048, 768
_F = 3072
_E = 8
_EPS = 1e-12
_T = 256
_G = _S * 2 // _T + _E
_P = _G * _T
_NC, _NS = 2, 16
_NW = _NC * _NS
_TPW = _S // _NW
_SQRT1_2 = 0.7071067811865476
_HW = _H // 2
_HIMASK = -65536


def _pack_rows(x):
    xb = x.astype(jnp.bfloat16).astype(jnp.float32)
    bits = lax.bitcast_convert_type(xb, jnp.int32)
    a = bits[:, :_HW]
    b = bits[:, _HW:]
    return lax.shift_right_logical(a, 16) | (b & _HIMASK)


def _unpack_rows(p):
    a = lax.bitcast_convert_type(lax.shift_left(p, 16), jnp.float32)
    b = lax.bitcast_convert_type(p & _HIMASK, jnp.float32)
    return jnp.concatenate([a, b], axis=1)


def _router_body(x_ref, rw_ref, rb_ref,
                 pos0_ref, pos1_ref, w0_ref, w1_ref, te_ref, xp_ref):
    x = x_ref[...]
    xp_ref[...] = _pack_rows(x)
    logits = lax.dot_general(
        x.astype(jnp.bfloat16), rw_ref[...].astype(jnp.bfloat16),
        (((1,), (1,)), ((), ())),
        preferred_element_type=jnp.float32) + rb_ref[...]
    iota_e = lax.broadcasted_iota(jnp.int32, logits.shape, 1)
    m0 = jnp.max(logits, axis=-1, keepdims=True)
    e0 = jnp.min(jnp.where(logits >= m0, iota_e, _E), axis=-1, keepdims=True)
    masked = jnp.where(iota_e == e0, -jnp.inf, logits)
    m1 = jnp.max(masked, axis=-1, keepdims=True)
    e1 = jnp.min(jnp.where(masked >= m1, iota_e, _E), axis=-1, keepdims=True)
    w0 = 1.0 / (1.0 + jnp.exp(m1 - m0))
    w0_ref[...] = w0
    w1_ref[...] = 1.0 - w0
    sel = ((iota_e == e0) | (iota_e == e1)).astype(jnp.bfloat16)
    tri = (lax.broadcasted_iota(jnp.int32, (_S, _S), 1)
           < lax.broadcasted_iota(jnp.int32, (_S, _S), 0)).astype(jnp.bfloat16)
    rank = lax.dot_general(tri, sel, (((1,), (0,)), ((), ())),
                           preferred_element_type=jnp.float32)
    count = jnp.sum(sel.astype(jnp.float32), axis=0, keepdims=True)
    pc = ((count.astype(jnp.int32) + _T - 1) // _T) * _T
    trie = (lax.broadcasted_iota(jnp.int32, (_E, _E), 0)
            < lax.broadcasted_iota(jnp.int32, (_E, _E), 1)).astype(jnp.float32)
    pstart = lax.dot_general(pc.astype(jnp.float32), trie,
                             (((1,), (0,)), ((), ())),
                             preferred_element_type=jnp.float32,
                             precision=lax.Precision.HIGHEST)
    rank0 = jnp.sum(jnp.where(iota_e == e0, rank, 0.0), axis=1, keepdims=True)
    rank1 = jnp.sum(jnp.where(iota_e == e1, rank, 0.0), axis=1, keepdims=True)
    ps0 = jnp.sum(jnp.where(iota_e == e0, pstart, 0.0), axis=1, keepdims=True)
    ps1 = jnp.sum(jnp.where(iota_e == e1, pstart, 0.0), axis=1, keepdims=True)
    pos0_ref[...] = (ps0 + rank0).astype(jnp.int32)
    pos1_ref[...] = (ps1 + rank1).astype(jnp.int32)
    psi = pstart.astype(jnp.int32)
    n_used = jnp.sum(pc, axis=1, keepdims=True) // _T
    iota_g = lax.broadcasted_iota(jnp.int32, (_G, _E), 0)
    gT = jnp.minimum(iota_g, n_used - 1) * _T
    te = jnp.sum((jnp.broadcast_to(psi, (_G, _E)) <= gT).astype(jnp.int32),
                 axis=1, keepdims=True) - 1
    te = jnp.clip(te, 0, _E - 1)
    iota_e_g = lax.broadcasted_iota(jnp.int32, (_G, _E), 1)
    used_b = jnp.broadcast_to((count > 0).astype(jnp.int32), (_G, _E))
    ordv = jnp.sum(jnp.where(iota_e_g < te, used_b, 0),
                   axis=1, keepdims=True)
    parity = ordv % 2
    te_prev = jnp.concatenate(
        [jnp.full((1, 1), -1, jnp.int32), te[:-1]], axis=0)
    is_start = (te != te_prev).astype(jnp.int32)
    nxt = jnp.min(jnp.where((used_b == 1) & (iota_e_g > te), iota_e_g, _E),
                  axis=1, keepdims=True)
    te_ref[...] = jnp.concatenate(
        [te, parity, is_start, nxt,
         jnp.broadcast_to(n_used, (8, 1))], axis=0)


def _router(flat, router_w, router_b):
    return pl.pallas_call(
        _router_body,
        grid=(1,),
        in_specs=[
            pl.BlockSpec((_S, _H), lambda i: (0, 0)),
            pl.BlockSpec((_E, _H), lambda i: (0, 0)),
            pl.BlockSpec((1, _E), lambda i: (0, 0)),
        ],
        out_specs=[
            pl.BlockSpec((_S, 1), lambda i: (0, 0)),
            pl.BlockSpec((_S, 1), lambda i: (0, 0)),
            pl.BlockSpec((_S, 1), lambda i: (0, 0)),
            pl.BlockSpec((_S, 1), lambda i: (0, 0)),
            pl.BlockSpec((4 * _G + 8, 1), lambda i: (0, 0)),
            pl.BlockSpec((_S, _HW), lambda i: (0, 0)),
        ],
        out_shape=[
            jax.ShapeDtypeStruct((_S, 1), jnp.int32),
            jax.ShapeDtypeStruct((_S, 1), jnp.int32),
            jax.ShapeDtypeStruct((_S, 1), jnp.float32),
            jax.ShapeDtypeStruct((_S, 1), jnp.float32),
            jax.ShapeDtypeStruct((4 * _G + 8, 1), jnp.int32),
            jax.ShapeDtypeStruct((_S, _HW), jnp.int32),
        ],
    )(flat, router_w, router_b.reshape(1, _E))


def _sc_scatter_body(flat_hbm, pos0_hbm, pos1_hbm, xs_hbm,
                     rows_v, i0_v, i1_v, sem, sem2):
    wid = lax.axis_index("s") * _NC + lax.axis_index("c")
    base = wid * _TPW
    cr = pltpu.async_copy(flat_hbm.at[pl.ds(base, _TPW)], rows_v, sem)
    c0 = pltpu.async_copy(pos0_hbm.at[pl.ds(base, _TPW)], i0_v, sem2)
    c1 = pltpu.async_copy(pos1_hbm.at[pl.ds(base, _TPW)], i1_v, sem2)
    cr.wait()
    c0.wait()
    c1.wait()
    s0 = pltpu.async_copy(rows_v, xs_hbm.at[i0_v], sem)
    s1 = pltpu.async_copy(rows_v, xs_hbm.at[i1_v], sem2)
    s0.wait()
    s1.wait()


def _sc_scatter(flat, pos0, pos1):
    mesh = plsc.VectorSubcoreMesh(core_axis_name="c", subcore_axis_name="s")
    return pl.kernel(
        _sc_scatter_body,
        out_type=jax.ShapeDtypeStruct((_P, _HW), jnp.int32),
        mesh=mesh,
        scratch_types=[
            pltpu.VMEM((_TPW, _HW), jnp.int32),
            pltpu.VMEM((_TPW,), jnp.int32),
            pltpu.VMEM((_TPW,), jnp.int32),
            pltpu.SemaphoreType.DMA,
            pltpu.SemaphoreType.DMA,
        ],
    )(flat, pos0, pos1)


def _ffn_body(m_ref, xs_ref, b1_ref, b2_ref, w1_hbm, w2_hbm, y_ref,
              w1buf, w2buf, semw):
    g = pl.program_id(0)
    e = m_ref[g]
    par = m_ref[_G + g]
    st = m_ref[2 * _G + g]
    nxe = m_ref[3 * _G + g]
    nu = m_ref[4 * _G]

    @pl.when(g == 0)
    def _():
        pltpu.make_async_copy(w1_hbm.at[e], w1buf.at[par], semw.at[par]).start()
        pltpu.make_async_copy(w2_hbm.at[e], w2buf.at[par], semw.at[par]).start()

    @pl.when(st == 1)
    def _():
        pltpu.make_async_copy(w1_hbm.at[e], w1buf.at[par], semw.at[par]).wait()
        pltpu.make_async_copy(w2_hbm.at[e], w2buf.at[par], semw.at[par]).wait()

    @pl.when((st == 1) & (nxe < _E))
    def _():
        pltpu.make_async_copy(
            w1_hbm.at[nxe], w1buf.at[1 - par], semw.at[1 - par]).start()
        pltpu.make_async_copy(
            w2_hbm.at[nxe], w2buf.at[1 - par], semw.at[1 - par]).start()

    @pl.when(g < nu)
    def _():
        xb = _unpack_rows(xs_ref[...]).astype(jnp.bfloat16)
        h1 = lax.dot_general(xb, w1buf[par].astype(jnp.bfloat16),
                             (((1,), (1,)), ((), ())),
                             preferred_element_type=jnp.float32) + b1_ref[0]
        h1 = 0.5 * h1 * (1.0 + lax.erf(h1 * _SQRT1_2))
        y_ref[...] = _pack_rows(lax.dot_general(
            h1.astype(jnp.bfloat16), w2buf[par].astype(jnp.bfloat16),
            (((1,), (1,)), ((), ())),
            preferred_element_type=jnp.float32) + b2_ref[0])


def _ffn(te_flat, xs, b1r, b2r, W1, W2):
    grid_spec = pltpu.PrefetchScalarGridSpec(
        num_scalar_prefetch=1,
        grid=(_G,),
        in_specs=[
            pl.BlockSpec((_T, _HW), lambda g, m: (g, 0)),
            pl.BlockSpec((1, 1, _F), lambda g, m: (m[g], 0, 0)),
            pl.BlockSpec((1, 1, _H), lambda g, m: (m[g], 0, 0)),
            pl.BlockSpec(memory_space=pltpu.MemorySpace.HBM),
            pl.BlockSpec(memory_space=pltpu.MemorySpace.HBM),
        ],
        out_specs=pl.BlockSpec((_T, _HW), lambda g, m: (g, 0)),
        scratch_shapes=[
            pltpu.VMEM((2, _F, _H), jnp.float32),
            pltpu.VMEM((2, _H, _F), jnp.float32),
            pltpu.SemaphoreType.DMA((2,)),
        ],
    )
    return pl.pallas_call(
        _ffn_body,
        grid_spec=grid_spec,
        out_shape=jax.ShapeDtypeStruct((_P, _HW), jnp.int32),
        compiler_params=pltpu.CompilerParams(
            vmem_limit_bytes=120 * 1024 * 1024),
    )(te_flat, xs, b1r, b2r, W1, W2)


def _sc_gather_body(y_hbm, pos0_hbm, pos1_hbm, y0_hbm, y1_hbm,
                    rows0_v, rows1_v, i0_v, i1_v, sem, sem2):
    wid = lax.axis_index("s") * _NC + lax.axis_index("c")
    base = wid * _TPW
    c0 = pltpu.async_copy(pos0_hbm.at[pl.ds(base, _TPW)], i0_v, sem)
    c1 = pltpu.async_copy(pos1_hbm.at[pl.ds(base, _TPW)], i1_v, sem2)
    c0.wait()
    c1.wait()
    g0 = pltpu.async_copy(y_hbm.at[i0_v], rows0_v, sem)
    g1 = pltpu.async_copy(y_hbm.at[i1_v], rows1_v, sem2)
    g0.wait()
    g1.wait()
    s0 = pltpu.async_copy(rows0_v, y0_hbm.at[pl.ds(base, _TPW)], sem)
    s1 = pltpu.async_copy(rows1_v, y1_hbm.at[pl.ds(base, _TPW)], sem2)
    s0.wait()
    s1.wait()


def _sc_gather(y, pos0, pos1):
    mesh = plsc.VectorSubcoreMesh(core_axis_name="c", subcore_axis_name="s")
    return pl.kernel(
        _sc_gather_body,
        out_type=[
            jax.ShapeDtypeStruct((_S, _HW), jnp.int32),
            jax.ShapeDtypeStruct((_S, _HW), jnp.int32),
        ],
        mesh=mesh,
        scratch_types=[
            pltpu.VMEM((_TPW, _HW), jnp.int32),
            pltpu.VMEM((_TPW, _HW), jnp.int32),
            pltpu.VMEM((_TPW,), jnp.int32),
            pltpu.VMEM((_TPW,), jnp.int32),
            pltpu.SemaphoreType.DMA,
            pltpu.SemaphoreType.DMA,
        ],
    )(y, pos0, pos1)


def _combine_body(x_ref, y0_ref, y1_ref, w0_ref, w1_ref, g_ref, b_ref,
                  out_ref):
    y0 = _unpack_rows(y0_ref[...])
    y1 = _unpack_rows(y1_ref[...])
    u = x_ref[...] + w0_ref[...] * y0 + w1_ref[...] * y1
    mu = jnp.mean(u, axis=-1, keepdims=True)
    var = jnp.mean((u - mu) ** 2, axis=-1, keepdims=True)
    out_ref[...] = (u - mu) * lax.rsqrt(var + _EPS) * g_ref[...] + b_ref[...]


def _combine(flat, y0, y1, w0, w1, ln_g, ln_b):
    tn = 256
    return pl.pallas_call(
        _combine_body,
        grid=(_S // tn,),
        in_specs=[
            pl.BlockSpec((tn, _H), lambda t: (t, 0)),
            pl.BlockSpec((tn, _HW), lambda t: (t, 0)),
            pl.BlockSpec((tn, _HW), lambda t: (t, 0)),
            pl.BlockSpec((tn, 1), lambda t: (t, 0)),
            pl.BlockSpec((tn, 1), lambda t: (t, 0)),
            pl.BlockSpec((1, _H), lambda t: (0, 0)),
            pl.BlockSpec((1, _H), lambda t: (0, 0)),
        ],
        out_specs=pl.BlockSpec((tn, _H), lambda t: (t, 0)),
        out_shape=jax.ShapeDtypeStruct((_S, _H), jnp.float32),
    )(flat, y0, y1, w0, w1, ln_g.reshape(1, _H), ln_b.reshape(1, _H))


def kernel(hidden_states, router_w, router_b, W1, b1, W2, b2, ln_g, ln_b):
    flat = hidden_states.reshape(_S, _H)
    pos0_2d, pos1_2d, w0, w1, te, flat_pk = _router(flat, router_w, router_b)
    pos0 = pos0_2d.reshape(_S)
    pos1 = pos1_2d.reshape(_S)
    xs = _sc_scatter(flat_pk, pos0, pos1)
    y = _ffn(te.reshape(4 * _G + 8), xs, b1.reshape(_E, 1, _F),
             b2.reshape(_E, 1, _H), W1, W2)
    y0, y1 = _sc_gather(y, pos0, pos1)
    out = _combine(flat, y0, y1, w0, w1, ln_g, ln_b)
    return out.reshape(_B, _S, _H)
